# Initial kernel scaffold; baseline (speedup 1.0000x reference)
import functools

import jax
import jax.numpy as jnp
from jax import lax
from jax.experimental import pallas as pl
from jax.experimental.pallas import tpu as pltpu
from jax.experimental.pallas import tpu_sc as plsc

N = 10000
NINC = 320000
NT = 16
RPT = N // NT
NCHUNK = 250
CH = 80

_mesh = lambda: plsc.VectorSubcoreMesh(core_axis_name="c", subcore_axis_name="s")


def _fill(ref, rows, value):
    ncol = ref.shape[1] // 16
    v = jnp.full((16,), value, jnp.float32)

    def body(r, _):
        for j in range(ncol):
            ref[r, pl.ds(j * 16, 16)] = v
        return 0

    lax.fori_loop(0, rows, body, 0)


def _degrees(nidx, eidx):
    @functools.partial(
        pl.kernel,
        out_type=(jax.ShapeDtypeStruct((N, 16), jnp.float32),
                  jax.ShapeDtypeStruct((N, 16), jnp.float32)),
        mesh=_mesh(),
        scratch_types=[
            pltpu.VMEM_SHARED((N, 16), jnp.float32),
            pltpu.VMEM((NCHUNK, CH), jnp.int32),
            pltpu.VMEM((CH, 16), jnp.float32),
            pltpu.VMEM((125, 16), jnp.float32),
            pltpu.VMEM((RPT, 16), jnp.float32),
        ],
    )
    def deg_kernel(nidx_h, eidx_h, de_o, dv_o, cnt, idxb, ones, zb, dbuf):
        c = lax.axis_index("c")
        s = lax.axis_index("s")
        r0 = s * RPT
        _fill(ones, CH, 1.0)
        _fill(zb, 125, 0.0)

        @pl.when(c == 0)
        def _():
            pltpu.sync_copy(eidx_h.at[s], idxb)

        @pl.when(c != 0)
        def _():
            pltpu.sync_copy(nidx_h.at[s], idxb)

        for k in range(RPT // 125):
            pltpu.sync_copy(zb, cnt.at[pl.ds(r0 + 125 * k, 125)])
        plsc.subcore_barrier()

        def body(j, _):
            pltpu.sync_copy(ones, cnt.at[idxb.at[j]], add=True)
            return 0

        lax.fori_loop(0, NCHUNK, body, 0)
        plsc.subcore_barrier()

        pltpu.sync_copy(cnt.at[pl.ds(r0, RPT)], dbuf)

        def inv(r, _):
            dbuf[r, :] = 1.0 / jnp.maximum(dbuf[r, :], 1.0)
            return 0

        lax.fori_loop(0, RPT, inv, 0)

        @pl.when(c == 0)
        def _():
            pltpu.sync_copy(dbuf, de_o.at[pl.ds(r0, RPT)])

        @pl.when(c != 0)
        def _():
            pltpu.sync_copy(dbuf, dv_o.at[pl.ds(r0, RPT)])

    return deg_kernel(nidx, eidx)


def _make_layer(w, relu):
    @functools.partial(
        pl.kernel,
        out_type=jax.ShapeDtypeStruct((2, N, w), jnp.float32),
        mesh=_mesh(),
        scratch_types=[
            pltpu.VMEM_SHARED((N, w), jnp.float32),
            pltpu.VMEM_SHARED((N, w), jnp.float32),
            pltpu.VMEM_SHARED((N, w), jnp.float32),
            pltpu.VMEM((NCHUNK, CH), jnp.int32),
            pltpu.VMEM((NCHUNK, CH), jnp.int32),
            pltpu.VMEM((CH, w), jnp.float32),
            pltpu.VMEM((125, w), jnp.float32),
            pltpu.VMEM((RPT, w), jnp.float32),
            pltpu.VMEM((RPT, 16), jnp.float32),
            pltpu.SemaphoreType.DMA,
        ],
    )
    def layer_kernel(x3, nidx_h, eidx_h, de16, dv16, out,
                     xs, yacc, zacc, nidx, eidx, gbuf, zb, nbuf, dbuf, sem):
        c = lax.axis_index("c")
        s = lax.axis_index("s")
        r0 = s * RPT

        _fill(zb, 125, 0.0)
        pltpu.sync_copy(nidx_h.at[s], nidx)
        pltpu.sync_copy(eidx_h.at[s], eidx)
        pltpu.sync_copy(x3.at[c].at[pl.ds(r0, RPT)], xs.at[pl.ds(r0, RPT)])
        for k in range(RPT // 125):
            pltpu.sync_copy(zb, yacc.at[pl.ds(r0 + 125 * k, 125)])
            pltpu.sync_copy(zb, zacc.at[pl.ds(r0 + 125 * k, 125)])
        plsc.subcore_barrier()

        def s1(j, _):
            pltpu.async_copy(xs.at[nidx.at[j]], gbuf, sem).wait()
            pltpu.sync_copy(gbuf, yacc.at[eidx.at[j]], add=True)
            return 0

        lax.fori_loop(0, NCHUNK, s1, 0)
        plsc.subcore_barrier()

        pltpu.sync_copy(de16.at[pl.ds(r0, RPT)], dbuf)
        pltpu.sync_copy(yacc.at[pl.ds(r0, RPT)], nbuf)

        def norm1(r, _):
            sv = dbuf[r, :]
            for j in range(w // 16):
                nbuf[r, pl.ds(j * 16, 16)] = nbuf[r, pl.ds(j * 16, 16)] * sv
            return 0

        lax.fori_loop(0, RPT, norm1, 0)
        pltpu.sync_copy(nbuf, yacc.at[pl.ds(r0, RPT)])
        plsc.subcore_barrier()

        def s2(j, _):
            pltpu.async_copy(yacc.at[eidx.at[j]], gbuf, sem).wait()
            pltpu.sync_copy(gbuf, zacc.at[nidx.at[j]], add=True)
            return 0

        lax.fori_loop(0, NCHUNK, s2, 0)
        plsc.subcore_barrier()

        pltpu.sync_copy(dv16.at[pl.ds(r0, RPT)], dbuf)
        pltpu.sync_copy(zacc.at[pl.ds(r0, RPT)], nbuf)

        def norm2(r, _):
            sv = dbuf[r, :]
            for j in range(w // 16):
                v = nbuf[r, pl.ds(j * 16, 16)] * sv
                if relu:
                    v = jnp.maximum(v, 0.0)
                nbuf[r, pl.ds(j * 16, 16)] = v
            return 0

        lax.fori_loop(0, RPT, norm2, 0)
        pltpu.sync_copy(nbuf, out.at[c].at[pl.ds(r0, RPT)])

    return layer_kernel


def _mm1(x, W1, b1):
    def body(x_ref, w_ref, b_ref, o_ref):
        acc = jnp.dot(x_ref[...], w_ref[...],
                      preferred_element_type=jnp.float32) + b_ref[...]
        o_ref[0] = acc[:, :64]
        o_ref[1] = acc[:, 64:]

    return pl.pallas_call(
        body,
        grid=(10,),
        in_specs=[pl.BlockSpec((1000, 128), lambda i: (i, 0)),
                  pl.BlockSpec((128, 128), lambda i: (0, 0)),
                  pl.BlockSpec((1, 128), lambda i: (0, 0))],
        out_specs=pl.BlockSpec((2, 1000, 64), lambda i: (0, i, 0)),
        out_shape=jax.ShapeDtypeStruct((2, N, 64), jnp.float32),
    )(x, W1, b1.reshape(1, 128))


def _mm2(z, W2, b2):
    def body(z_ref, w_ref, b_ref, o_ref):
        acc = (jnp.dot(z_ref[0], w_ref[0], preferred_element_type=jnp.float32)
               + jnp.dot(z_ref[1], w_ref[1], preferred_element_type=jnp.float32)
               + b_ref[...])
        o_ref[0] = acc[:, :32]
        o_ref[1] = acc[:, 32:]

    return pl.pallas_call(
        body,
        grid=(10,),
        in_specs=[pl.BlockSpec((2, 1000, 64), lambda i: (0, i, 0)),
                  pl.BlockSpec((2, 64, 64), lambda i: (0, 0, 0)),
                  pl.BlockSpec((1, 64), lambda i: (0, 0))],
        out_specs=pl.BlockSpec((2, 1000, 32), lambda i: (0, i, 0)),
        out_shape=jax.ShapeDtypeStruct((2, N, 32), jnp.float32),
    )(z, W2.reshape(2, 64, 64), b2.reshape(1, 64))


_layer64 = _make_layer(64, relu=True)
_layer32 = _make_layer(32, relu=False)


def kernel(x, hyperedge_index, W1, b1, W2, b2):
    nidx = hyperedge_index[0].reshape(NT, NCHUNK, CH)
    eidx = hyperedge_index[1].reshape(NT, NCHUNK, CH)
    de16, dv16 = _degrees(nidx, eidx)
    x1 = _mm1(x, W1, b1)
    z1 = _layer64(x1, nidx, eidx, de16, dv16)
    x2 = _mm2(z1, W2, b2)
    z2 = _layer32(x2, nidx, eidx, de16, dv16)
    return jnp.concatenate([z2[0], z2[1]], axis=1)


# speedup vs baseline: 6.0446x; 6.0446x over previous
import functools

import jax
import jax.numpy as jnp
from jax import lax
from jax.experimental import pallas as pl
from jax.experimental.pallas import tpu as pltpu
from jax.experimental.pallas import tpu_sc as plsc

N = 10000
NP = 10240
NINC = 320000
NT = 16
RPT = NP // NT
ZR = 128
NCHUNK = 250
NCB = 50
CH = 80

_mesh = lambda: plsc.VectorSubcoreMesh(core_axis_name="c", subcore_axis_name="s")
_SC_PARAMS = pltpu.CompilerParams(use_tc_tiling_on_sc=False)


def _fill(ref, rows, value):
    ncol = ref.shape[1] // 16
    v = jnp.full((16,), value, jnp.float32)

    def body(r, _):
        for j in range(ncol):
            ref[r, pl.ds(j * 16, 16)] = v
        return 0

    lax.fori_loop(0, rows, body, 0)


def _degrees(nidx, eidx):
    @functools.partial(
        pl.kernel,
        out_type=(jax.ShapeDtypeStruct((NP, 16), jnp.float32),
                  jax.ShapeDtypeStruct((NP, 16), jnp.float32)),
        mesh=_mesh(),
        compiler_params=_SC_PARAMS,
        scratch_types=[
            pltpu.VMEM_SHARED((NP, 16), jnp.float32),
            pltpu.VMEM((NCHUNK, CH), jnp.int32),
            pltpu.VMEM((CH, 16), jnp.float32),
            pltpu.VMEM((ZR, 16), jnp.float32),
            pltpu.VMEM((RPT, 16), jnp.float32),
        ],
    )
    def deg_kernel(nidx_h, eidx_h, de_o, dv_o, cnt, idxb, ones, zb, dbuf):
        c = lax.axis_index("c")
        s = lax.axis_index("s")
        r0 = s * RPT
        _fill(ones, CH, 1.0)
        _fill(zb, ZR, 0.0)

        @pl.when(c == 0)
        def _():
            pltpu.sync_copy(eidx_h.at[s], idxb)

        @pl.when(c != 0)
        def _():
            pltpu.sync_copy(nidx_h.at[s], idxb)

        for k in range(RPT // ZR):
            pltpu.sync_copy(zb, cnt.at[pl.ds(r0 + ZR * k, ZR)])
        plsc.subcore_barrier()

        def body(j, _):
            pltpu.sync_copy(ones, cnt.at[idxb.at[j]], add=True)
            return 0

        lax.fori_loop(0, NCHUNK, body, 0)
        plsc.subcore_barrier()

        pltpu.sync_copy(cnt.at[pl.ds(r0, RPT)], dbuf)

        def inv(r, _):
            dbuf[r, :] = 1.0 / jnp.maximum(dbuf[r, :], 1.0)
            return 0

        lax.fori_loop(0, RPT, inv, 0)

        @pl.when(c == 0)
        def _():
            pltpu.sync_copy(dbuf, de_o.at[pl.ds(r0, RPT)])

        @pl.when(c != 0)
        def _():
            pltpu.sync_copy(dbuf, dv_o.at[pl.ds(r0, RPT)])

    return deg_kernel(nidx, eidx)


def _make_layer(w, relu):
    @functools.partial(
        pl.kernel,
        out_type=jax.ShapeDtypeStruct((2, NP, w), jnp.float32),
        mesh=_mesh(),
        compiler_params=_SC_PARAMS,
        scratch_types=[
            pltpu.VMEM_SHARED((NP, w), jnp.float32),
            pltpu.VMEM_SHARED((NP, w), jnp.float32),
            pltpu.VMEM((NCB, CH), jnp.int32),
            pltpu.VMEM((NCB, CH), jnp.int32),
            pltpu.VMEM((CH, w), jnp.float32),
            pltpu.VMEM((CH, 16), jnp.float32),
            pltpu.SemaphoreType.DMA,
        ],
    )
    def layer_kernel(x2h, nidx_h, eidx_h, de16, dv16, out,
                     yacc, zacc, gidx, sidx, gbuf, dbuf, sem):
        c = lax.axis_index("c")
        s = lax.axis_index("s")
        r0 = s * RPT
        offv = jnp.broadcast_to(c * NP, (16,)).astype(jnp.int32)

        _fill(gbuf, CH, 0.0)
        for k in range(RPT // CH):
            pltpu.sync_copy(gbuf, yacc.at[pl.ds(r0 + CH * k, CH)])
            pltpu.sync_copy(gbuf, zacc.at[pl.ds(r0 + CH * k, CH)])
        plsc.subcore_barrier()

        def load_block(b, idx_h, dst, offset):
            pltpu.sync_copy(idx_h.at[s].at[pl.ds(b * NCB, NCB)], dst)
            if offset:
                def add_off(r, _):
                    for k in range(CH // 16):
                        dst[r, pl.ds(k * 16, 16)] = (
                            dst[r, pl.ds(k * 16, 16)] + offv)
                    return 0
                lax.fori_loop(0, NCB, add_off, 0)

        def blk1(b, _):
            load_block(b, nidx_h, gidx, True)
            load_block(b, eidx_h, sidx, False)

            def s1(j, _):
                pltpu.async_copy(x2h.at[gidx.at[j]], gbuf, sem).wait()
                pltpu.sync_copy(gbuf, yacc.at[sidx.at[j]], add=True)
                return 0

            lax.fori_loop(0, NCB, s1, 0)
            return 0

        lax.fori_loop(0, NCHUNK // NCB, blk1, 0)
        plsc.subcore_barrier()

        def norm_slabs(dsrc, acc, dst, do_relu):
            def slab(k, _):
                q0 = r0 + CH * k
                pltpu.sync_copy(dsrc.at[pl.ds(q0, CH)], dbuf)
                pltpu.sync_copy(acc.at[pl.ds(q0, CH)], gbuf)

                def row(r, _):
                    sv = dbuf[r, :]
                    for j in range(w // 16):
                        v = gbuf[r, pl.ds(j * 16, 16)] * sv
                        if do_relu:
                            v = jnp.maximum(v, 0.0)
                        gbuf[r, pl.ds(j * 16, 16)] = v
                    return 0

                lax.fori_loop(0, CH, row, 0)
                pltpu.sync_copy(gbuf, dst.at[pl.ds(q0, CH)])
                return 0

            lax.fori_loop(0, RPT // CH, slab, 0)

        norm_slabs(de16, yacc, yacc, False)
        plsc.subcore_barrier()

        def blk2(b, _):
            load_block(b, eidx_h, gidx, False)
            load_block(b, nidx_h, sidx, False)

            def s2(j, _):
                pltpu.async_copy(yacc.at[gidx.at[j]], gbuf, sem).wait()
                pltpu.sync_copy(gbuf, zacc.at[sidx.at[j]], add=True)
                return 0

            lax.fori_loop(0, NCB, s2, 0)
            return 0

        lax.fori_loop(0, NCHUNK // NCB, blk2, 0)
        plsc.subcore_barrier()

        norm_slabs(dv16, zacc, out.at[c], relu)

    return layer_kernel


def _mm1(x, W1, b1):
    def body(x_ref, w_ref, b_ref, o_ref):
        acc = jnp.dot(x_ref[...], w_ref[...],
                      preferred_element_type=jnp.float32) + b_ref[...]
        o_ref[0] = acc[:, :64]
        o_ref[1] = acc[:, 64:]

    return pl.pallas_call(
        body,
        grid=(10,),
        in_specs=[pl.BlockSpec((1000, 128), lambda i: (i, 0)),
                  pl.BlockSpec((128, 128), lambda i: (0, 0)),
                  pl.BlockSpec((1, 128), lambda i: (0, 0))],
        out_specs=pl.BlockSpec((2, 1000, 64), lambda i: (0, i, 0)),
        out_shape=jax.ShapeDtypeStruct((2, NP, 64), jnp.float32),
    )(x, W1, b1.reshape(1, 128))


def _mm2(z, W2, b2):
    def body(z_ref, w_ref, b_ref, o_ref):
        acc = (jnp.dot(z_ref[0], w_ref[0], preferred_element_type=jnp.float32)
               + jnp.dot(z_ref[1], w_ref[1], preferred_element_type=jnp.float32)
               + b_ref[...])
        o_ref[0] = acc[:, :32]
        o_ref[1] = acc[:, 32:]

    return pl.pallas_call(
        body,
        grid=(10,),
        in_specs=[pl.BlockSpec((2, 1000, 64), lambda i: (0, i, 0)),
                  pl.BlockSpec((2, 64, 64), lambda i: (0, 0, 0)),
                  pl.BlockSpec((1, 64), lambda i: (0, 0))],
        out_specs=pl.BlockSpec((2, 1000, 32), lambda i: (0, i, 0)),
        out_shape=jax.ShapeDtypeStruct((2, NP, 32), jnp.float32),
    )(z, W2.reshape(2, 64, 64), b2.reshape(1, 64))


_layer64 = _make_layer(64, relu=True)
_layer32 = _make_layer(32, relu=False)


def kernel(x, hyperedge_index, W1, b1, W2, b2):
    nidx = hyperedge_index[0].reshape(NT, NCHUNK, CH)
    eidx = hyperedge_index[1].reshape(NT, NCHUNK, CH)
    de16, dv16 = _degrees(nidx, eidx)
    x1 = _mm1(x, W1, b1)
    z1 = _layer64(x1.reshape(2 * NP, 64), nidx, eidx, de16, dv16)
    x2 = _mm2(z1, W2, b2)
    z2 = _layer32(x2.reshape(2 * NP, 32), nidx, eidx, de16, dv16)
    return jnp.concatenate([z2[0, :N], z2[1, :N]], axis=1)
